# speedup vs baseline: 4.8494x; 4.8494x over previous
import functools

import jax
import jax.numpy as jnp
from jax import lax
from jax.experimental import pallas as pl
from jax.experimental.pallas import tpu as pltpu
from jax.experimental.pallas import tpu_sc as plsc

BATCH = 4
SEQ = 8192
DIM = 128
TOTAL = BATCH * SEQ

NC = 2
NS = 16
NW = NC * NS
B_PER_W = TOTAL // NW
CHUNK = 128
NCHUNK = B_PER_W // CHUNK

_mesh = plsc.VectorSubcoreMesh(core_axis_name="c", subcore_axis_name="s")


@functools.partial(
    pl.kernel,
    mesh=_mesh,
    out_type=(
        jax.ShapeDtypeStruct((TOTAL, DIM), jnp.float32),
        jax.ShapeDtypeStruct((TOTAL, DIM), jnp.float32),
    ),
    scratch_types=[
        pltpu.VMEM((NCHUNK, CHUNK), jnp.int32),
        pltpu.VMEM((CHUNK, DIM), jnp.float32),
        pltpu.VMEM((CHUNK, DIM), jnp.float32),
        pltpu.SemaphoreType.DMA,
        pltpu.SemaphoreType.DMA,
    ],
)
def _gather_kernel(idx_hbm, cos_hbm, sin_hbm, cos_out, sin_out,
                   idx_v, cbuf, sbuf, csem, ssem):
    wid = lax.axis_index("s") * NC + lax.axis_index("c")
    pltpu.sync_copy(idx_hbm.at[wid], idx_v)
    for k in range(NCHUNK):
        base = wid * B_PER_W + k * CHUNK
        cg = pltpu.async_copy(cos_hbm.at[idx_v.at[k]], cbuf, csem)
        sg = pltpu.async_copy(sin_hbm.at[idx_v.at[k]], sbuf, ssem)
        cg.wait()
        sg.wait()
        pltpu.sync_copy(cbuf, cos_out.at[pl.ds(base, CHUNK)])
        pltpu.sync_copy(sbuf, sin_out.at[pl.ds(base, CHUNK)])


def kernel(position_ids, cos_cached, sin_cached):
    idx = position_ids.reshape(NW, NCHUNK, CHUNK)
    cos, sin = _gather_kernel(idx, cos_cached, sin_cached)
    return cos.reshape(BATCH, SEQ, DIM), sin.reshape(BATCH, SEQ, DIM)


# speedup vs baseline: 5.1197x; 1.0557x over previous
import functools

import jax
import jax.numpy as jnp
from jax import lax
from jax.experimental import pallas as pl
from jax.experimental.pallas import tpu as pltpu
from jax.experimental.pallas import tpu_sc as plsc

BATCH = 4
SEQ = 8192
DIM = 128
TOTAL = BATCH * SEQ

NC = 2
NS = 16
NW = NC * NS
B_PER_W = TOTAL // NW
CHUNK = 128
NCHUNK = B_PER_W // CHUNK

_mesh = plsc.VectorSubcoreMesh(core_axis_name="c", subcore_axis_name="s")


@functools.partial(
    pl.kernel,
    mesh=_mesh,
    out_type=(
        jax.ShapeDtypeStruct((TOTAL, DIM), jnp.float32),
        jax.ShapeDtypeStruct((TOTAL, DIM), jnp.float32),
    ),
    scratch_types=[
        pltpu.VMEM((NCHUNK, CHUNK), jnp.int32),
        pltpu.VMEM((2, CHUNK, DIM), jnp.float32),
        pltpu.VMEM((2, CHUNK, DIM), jnp.float32),
        pltpu.SemaphoreType.DMA,
        pltpu.SemaphoreType.DMA,
        pltpu.SemaphoreType.DMA,
        pltpu.SemaphoreType.DMA,
    ],
)
def _gather_kernel(idx_hbm, cos_hbm, sin_hbm, cos_out, sin_out,
                   idx_v, cbuf, sbuf, cgs, sgs, cws, sws):
    wid = lax.axis_index("s") * NC + lax.axis_index("c")
    pltpu.sync_copy(idx_hbm.at[wid], idx_v)
    cg = [None] * NCHUNK
    sg = [None] * NCHUNK
    cw = [None] * NCHUNK
    sw = [None] * NCHUNK
    cg[0] = pltpu.async_copy(cos_hbm.at[idx_v.at[0]], cbuf.at[0], cgs)
    sg[0] = pltpu.async_copy(sin_hbm.at[idx_v.at[0]], sbuf.at[0], sgs)
    for k in range(NCHUNK):
        slot = k % 2
        base = wid * B_PER_W + k * CHUNK
        cg[k].wait()
        sg[k].wait()
        if k + 1 < NCHUNK:
            if k >= 1:
                cw[k - 1].wait()
                sw[k - 1].wait()
            nslot = (k + 1) % 2
            cg[k + 1] = pltpu.async_copy(
                cos_hbm.at[idx_v.at[k + 1]], cbuf.at[nslot], cgs)
            sg[k + 1] = pltpu.async_copy(
                sin_hbm.at[idx_v.at[k + 1]], sbuf.at[nslot], sgs)
        cw[k] = pltpu.async_copy(
            cbuf.at[slot], cos_out.at[pl.ds(base, CHUNK)], cws)
        sw[k] = pltpu.async_copy(
            sbuf.at[slot], sin_out.at[pl.ds(base, CHUNK)], sws)
    cw[NCHUNK - 2].wait()
    sw[NCHUNK - 2].wait()
    cw[NCHUNK - 1].wait()
    sw[NCHUNK - 1].wait()


def kernel(position_ids, cos_cached, sin_cached):
    idx = position_ids.reshape(NW, NCHUNK, CHUNK)
    cos, sin = _gather_kernel(idx, cos_cached, sin_cached)
    return cos.reshape(BATCH, SEQ, DIM), sin.reshape(BATCH, SEQ, DIM)


# speedup vs baseline: 5.2484x; 1.0251x over previous
import functools

import jax
import jax.numpy as jnp
from jax import lax
from jax.experimental import pallas as pl
from jax.experimental.pallas import tpu as pltpu
from jax.experimental.pallas import tpu_sc as plsc

BATCH = 4
SEQ = 8192
DIM = 128
TOTAL = BATCH * SEQ

NC = 2
NS = 16
NW = NC * NS
B_PER_W = TOTAL // NW
CHUNK = 128
NCHUNK = B_PER_W // CHUNK
NBUF = 3

_mesh = plsc.VectorSubcoreMesh(core_axis_name="c", subcore_axis_name="s")


@functools.partial(
    pl.kernel,
    mesh=_mesh,
    out_type=(
        jax.ShapeDtypeStruct((TOTAL, DIM), jnp.float32),
        jax.ShapeDtypeStruct((TOTAL, DIM), jnp.float32),
    ),
    scratch_types=[
        pltpu.VMEM((NCHUNK, CHUNK), jnp.int32),
        pltpu.VMEM((NBUF, CHUNK, DIM), jnp.float32),
        pltpu.VMEM((NBUF, CHUNK, DIM), jnp.float32),
        pltpu.SemaphoreType.DMA,
        pltpu.SemaphoreType.DMA,
        pltpu.SemaphoreType.DMA,
        pltpu.SemaphoreType.DMA,
    ],
)
def _gather_kernel(idx_hbm, cos_hbm, sin_hbm, cos_out, sin_out,
                   idx_v, cbuf, sbuf, cgs, sgs, cws, sws):
    wid = lax.axis_index("s") * NC + lax.axis_index("c")
    pltpu.sync_copy(idx_hbm.at[wid], idx_v)
    cg = [None] * NCHUNK
    sg = [None] * NCHUNK
    cw = [None] * NCHUNK
    sw = [None] * NCHUNK
    for k in range(2):
        cg[k] = pltpu.async_copy(cos_hbm.at[idx_v.at[k]], cbuf.at[k], cgs)
        sg[k] = pltpu.async_copy(sin_hbm.at[idx_v.at[k]], sbuf.at[k], sgs)
    for k in range(NCHUNK):
        slot = k % NBUF
        base = wid * B_PER_W + k * CHUNK
        cg[k].wait()
        sg[k].wait()
        if k + 2 < NCHUNK:
            if k >= 1:
                cw[k - 1].wait()
                sw[k - 1].wait()
            nslot = (k + 2) % NBUF
            cg[k + 2] = pltpu.async_copy(
                cos_hbm.at[idx_v.at[k + 2]], cbuf.at[nslot], cgs)
            sg[k + 2] = pltpu.async_copy(
                sin_hbm.at[idx_v.at[k + 2]], sbuf.at[nslot], sgs)
        rows = pl.ds(base, CHUNK)
        cw[k] = pltpu.async_copy(cbuf.at[slot], cos_out.at[rows], cws)
        sw[k] = pltpu.async_copy(sbuf.at[slot], sin_out.at[rows], sws)
    for k in range(NCHUNK - 3, NCHUNK):
        cw[k].wait()
        sw[k].wait()


def kernel(position_ids, cos_cached, sin_cached):
    idx = position_ids.reshape(NW, NCHUNK, CHUNK)
    cos, sin = _gather_kernel(idx, cos_cached, sin_cached)
    return cos.reshape(BATCH, SEQ, DIM), sin.reshape(BATCH, SEQ, DIM)
